# baseline (speedup 1.0000x reference)
import functools

import jax
import jax.numpy as jnp
from jax import lax
from jax.experimental import pallas as pl
from jax.experimental.pallas import tpu as pltpu
from jax.experimental.pallas import tpu_sc as plsc

_B, _D, _H, _W = 2, 128, 512, 512
_HS = 512
_TH = 32

_NC, _NS, _L = 2, 16, 16
_NW = _NC * _NS
_RI = 4
_DC = 8
_NBUF = 4
_NCH = _D // _DC
_SC_ROWS = _B * _HS
_ROWS_PER_W = _SC_ROWS // _NW
_ITEMS = _ROWS_PER_W // _RI
_STEPS = _ITEMS * _NCH
_INF = float("inf")


def _compute_chunk(buf, m1s, m2s):
    for r in range(_RI):
        def col_body(jj, _, r=r):
            c0 = jj * 32
            m1a = m1s[r, pl.ds(c0, _L)]
            m2a = m2s[r, pl.ds(c0, _L)]
            m1b = m1s[r, pl.ds(c0 + _L, _L)]
            m2b = m2s[r, pl.ds(c0 + _L, _L)]
            for dd in range(_DC):
                xa = buf[dd, r, pl.ds(c0, _L)]
                xb = buf[dd, r, pl.ds(c0 + _L, _L)]
                m2a = jnp.minimum(m2a, jnp.maximum(m1a, xa))
                m1a = jnp.minimum(m1a, xa)
                m2b = jnp.minimum(m2b, jnp.maximum(m1b, xb))
                m1b = jnp.minimum(m1b, xb)
            m1s[r, pl.ds(c0, _L)] = m1a
            m2s[r, pl.ds(c0, _L)] = m2a
            m1s[r, pl.ds(c0 + _L, _L)] = m1b
            m2s[r, pl.ds(c0 + _L, _L)] = m2b
            return _

        lax.fori_loop(0, _W // 32, col_body, None)


def _sc_body(vol, out, buf0, buf1, buf2, buf3, m1s, m2s, obuf,
             sem0, sem1, sem2, sem3):
    wid = lax.axis_index("s") * _NC + lax.axis_index("c")
    row0 = wid * _ROWS_PER_W
    b = row0 // _HS
    h0 = row0 % _HS
    bufs = (buf0, buf1, buf2, buf3)
    sems = (sem0, sem1, sem2, sem3)

    def src(step):
        item = step // _NCH
        chunk = step % _NCH
        return vol.at[b, pl.ds(chunk * _DC, _DC), pl.ds(h0 + item * _RI, _RI), :]

    def start(step, k):
        pltpu.async_copy(src(step), bufs[k], sems[k])

    def wait(step, k):
        pltpu.make_async_copy(src(step), bufs[k], sems[k]).wait()

    def process(step, k):
        item = step // _NCH
        chunk = step % _NCH

        @pl.when(chunk == 0)
        def _():
            inf = jnp.full((_L,), _INF, jnp.float32)
            for r in range(_RI):
                def init_body(jj, _, r=r):
                    m1s[r, pl.ds(jj * _L, _L)] = inf
                    m2s[r, pl.ds(jj * _L, _L)] = inf
                    return _
                lax.fori_loop(0, _W // _L, init_body, None)

        wait(step, k)
        _compute_chunk(bufs[k], m1s, m2s)

        @pl.when(chunk == _NCH - 1)
        def _():
            for r in range(_RI):
                def out_body(jj, _, r=r):
                    sl = pl.ds(jj * _L, _L)
                    obuf[r, sl] = m2s[r, sl] - m1s[r, sl]
                    return _
                lax.fori_loop(0, _W // _L, out_body, None)
            pltpu.sync_copy(obuf, out.at[pl.ds(row0 + item * _RI, _RI), :])

    for k in range(_NBUF - 1):
        start(k, k)

    def loop_body(i, _):
        s0 = _NBUF * i
        for k in range(_NBUF):
            nxt = s0 + k + _NBUF - 1

            @pl.when(nxt < _STEPS)
            def _(nxt=nxt, k=k):
                start(nxt, (k + _NBUF - 1) % _NBUF)

            process(s0 + k, k)
        return _

    lax.fori_loop(0, _STEPS // _NBUF, loop_body, None)


def _sc_call(volume):
    mesh = plsc.VectorSubcoreMesh(core_axis_name="c", subcore_axis_name="s")
    return pl.kernel(
        _sc_body,
        out_type=jax.ShapeDtypeStruct((_SC_ROWS, _W), jnp.float32),
        mesh=mesh,
        scratch_types=(
            [pltpu.VMEM((_DC, _RI, _W), jnp.float32)] * _NBUF
            + [pltpu.VMEM((_RI, _W), jnp.float32)] * 3
            + [pltpu.SemaphoreType.DMA] * _NBUF
        ),
    )(volume)


def _tc_body(v_ref, o_ref):
    d = v_ref.shape[1]
    a = v_ref[0, 0]
    b = v_ref[0, 1]
    m1 = jnp.minimum(a, b)
    m2 = jnp.maximum(a, b)
    for i in range(2, d):
        x = v_ref[0, i]
        m2 = jnp.minimum(m2, jnp.maximum(m1, x))
        m1 = jnp.minimum(m1, x)
    o_ref[0, 0] = m2 - m1


def _tc_call(volume):
    nh = (_H - _HS) // _TH
    off = _HS // _TH
    return pl.pallas_call(
        _tc_body,
        grid=(_B, nh),
        in_specs=[pl.BlockSpec((1, _D, _TH, _W), lambda i, j: (i, 0, j + off, 0))],
        out_specs=pl.BlockSpec((1, 1, _TH, _W), lambda i, j: (i, 0, j, 0)),
        out_shape=jax.ShapeDtypeStruct((_B, 1, _H - _HS, _W), volume.dtype),
    )(volume)


@jax.jit
def _hybrid(volume):
    sc = _sc_call(volume).reshape(_B, 1, _HS, _W)
    if _HS == _H:
        return sc
    tc = _tc_call(volume)
    return jnp.concatenate([sc, tc], axis=2)


def kernel(volume):
    return _hybrid(volume)


# speedup vs baseline: 1.6127x; 1.6127x over previous
import functools

import jax
import jax.numpy as jnp
from jax import lax
from jax.experimental import pallas as pl
from jax.experimental.pallas import tpu as pltpu
from jax.experimental.pallas import tpu_sc as plsc

_B, _D, _H, _W = 2, 128, 512, 512
_HS = 128
_TH = 32

_NC, _NS, _L = 2, 16, 16
_NW = _NC * _NS
_RI = 4
_DC = 16
_NBUF = 2
_NCH = _D // _DC
_SC_ROWS = _B * _HS
_ROWS_PER_W = _SC_ROWS // _NW
_ITEMS = _ROWS_PER_W // _RI
_STEPS = _ITEMS * _NCH
_INF = float("inf")


def _compute_chunk(buf, m1s, m2s):
    for r in range(_RI):
        def col_body(jj, _, r=r):
            c0 = jj * 32
            m1a = m1s[r, pl.ds(c0, _L)]
            m2a = m2s[r, pl.ds(c0, _L)]
            m1b = m1s[r, pl.ds(c0 + _L, _L)]
            m2b = m2s[r, pl.ds(c0 + _L, _L)]
            for dd in range(_DC):
                xa = buf[dd, r, pl.ds(c0, _L)]
                xb = buf[dd, r, pl.ds(c0 + _L, _L)]
                m2a = jnp.minimum(m2a, jnp.maximum(m1a, xa))
                m1a = jnp.minimum(m1a, xa)
                m2b = jnp.minimum(m2b, jnp.maximum(m1b, xb))
                m1b = jnp.minimum(m1b, xb)
            m1s[r, pl.ds(c0, _L)] = m1a
            m2s[r, pl.ds(c0, _L)] = m2a
            m1s[r, pl.ds(c0 + _L, _L)] = m1b
            m2s[r, pl.ds(c0 + _L, _L)] = m2b
            return _

        lax.fori_loop(0, _W // 32, col_body, None)


def _sc_body(vol, out, *scratch):
    bufs = scratch[:_NBUF]
    m1s, m2s, obuf = scratch[_NBUF:_NBUF + 3]
    sems = scratch[_NBUF + 3:]
    wid = lax.axis_index("s") * _NC + lax.axis_index("c")
    row0 = wid * _ROWS_PER_W
    b = row0 // _HS
    h0 = row0 % _HS

    def src(step):
        item = step // _NCH
        chunk = step % _NCH
        return vol.at[b, pl.ds(chunk * _DC, _DC), pl.ds(h0 + item * _RI, _RI), :]

    def start(step, k):
        pltpu.async_copy(src(step), bufs[k], sems[k])

    def wait(step, k):
        pltpu.make_async_copy(src(step), bufs[k], sems[k]).wait()

    def process(step, k):
        item = step // _NCH
        chunk = step % _NCH

        @pl.when(chunk == 0)
        def _():
            inf = jnp.full((_L,), _INF, jnp.float32)
            for r in range(_RI):
                def init_body(jj, _, r=r):
                    m1s[r, pl.ds(jj * _L, _L)] = inf
                    m2s[r, pl.ds(jj * _L, _L)] = inf
                    return _
                lax.fori_loop(0, _W // _L, init_body, None)

        wait(step, k)
        _compute_chunk(bufs[k], m1s, m2s)

        @pl.when(chunk == _NCH - 1)
        def _():
            for r in range(_RI):
                def out_body(jj, _, r=r):
                    sl = pl.ds(jj * _L, _L)
                    obuf[r, sl] = m2s[r, sl] - m1s[r, sl]
                    return _
                lax.fori_loop(0, _W // _L, out_body, None)
            pltpu.sync_copy(obuf, out.at[pl.ds(row0 + item * _RI, _RI), :])

    for k in range(_NBUF - 1):
        start(k, k)

    def loop_body(i, _):
        s0 = _NBUF * i
        for k in range(_NBUF):
            nxt = s0 + k + _NBUF - 1

            @pl.when(nxt < _STEPS)
            def _(nxt=nxt, k=k):
                start(nxt, (k + _NBUF - 1) % _NBUF)

            process(s0 + k, k)
        return _

    lax.fori_loop(0, _STEPS // _NBUF, loop_body, None)


def _sc_call(volume):
    mesh = plsc.VectorSubcoreMesh(core_axis_name="c", subcore_axis_name="s")
    return pl.kernel(
        _sc_body,
        out_type=jax.ShapeDtypeStruct((_SC_ROWS, _W), jnp.float32),
        mesh=mesh,
        scratch_types=(
            [pltpu.VMEM((_DC, _RI, _W), jnp.float32)] * _NBUF
            + [pltpu.VMEM((_RI, _W), jnp.float32)] * 3
            + [pltpu.SemaphoreType.DMA] * _NBUF
        ),
    )(volume)


def _tc_body(v_ref, o_ref):
    d = v_ref.shape[1]
    a = v_ref[0, 0]
    b = v_ref[0, 1]
    m1 = jnp.minimum(a, b)
    m2 = jnp.maximum(a, b)
    for i in range(2, d):
        x = v_ref[0, i]
        m2 = jnp.minimum(m2, jnp.maximum(m1, x))
        m1 = jnp.minimum(m1, x)
    o_ref[0, 0] = m2 - m1


def _tc_call(volume):
    nh = (_H - _HS) // _TH
    off = _HS // _TH
    return pl.pallas_call(
        _tc_body,
        grid=(_B, nh),
        in_specs=[pl.BlockSpec((1, _D, _TH, _W), lambda i, j: (i, 0, j + off, 0))],
        out_specs=pl.BlockSpec((1, 1, _TH, _W), lambda i, j: (i, 0, j, 0)),
        out_shape=jax.ShapeDtypeStruct((_B, 1, _H - _HS, _W), volume.dtype),
    )(volume)


@jax.jit
def _hybrid(volume):
    sc = _sc_call(volume).reshape(_B, 1, _HS, _W)
    if _HS == _H:
        return sc
    tc = _tc_call(volume)
    return jnp.concatenate([sc, tc], axis=2)


def kernel(volume):
    return _hybrid(volume)


# speedup vs baseline: 1.6130x; 1.0001x over previous
import functools

import jax
import jax.numpy as jnp
from jax import lax
from jax.experimental import pallas as pl
from jax.experimental.pallas import tpu as pltpu
from jax.experimental.pallas import tpu_sc as plsc

_B, _D, _H, _W = 2, 128, 512, 512
_HS = 128
_TH = 32

_NC, _NS, _L = 2, 16, 16
_NW = _NC * _NS
_RI = 8
_DC = 8
_NBUF = 2
_NCH = _D // _DC
_SC_ROWS = _B * _HS
_ROWS_PER_W = _SC_ROWS // _NW
_ITEMS = _ROWS_PER_W // _RI
_STEPS = _ITEMS * _NCH
_INF = float("inf")


def _compute_chunk(buf, m1s, m2s):
    for r in range(_RI):
        def col_body(jj, _, r=r):
            c0 = jj * 32
            m1a = m1s[r, pl.ds(c0, _L)]
            m2a = m2s[r, pl.ds(c0, _L)]
            m1b = m1s[r, pl.ds(c0 + _L, _L)]
            m2b = m2s[r, pl.ds(c0 + _L, _L)]
            for dd in range(_DC):
                xa = buf[dd, r, pl.ds(c0, _L)]
                xb = buf[dd, r, pl.ds(c0 + _L, _L)]
                m2a = jnp.minimum(m2a, jnp.maximum(m1a, xa))
                m1a = jnp.minimum(m1a, xa)
                m2b = jnp.minimum(m2b, jnp.maximum(m1b, xb))
                m1b = jnp.minimum(m1b, xb)
            m1s[r, pl.ds(c0, _L)] = m1a
            m2s[r, pl.ds(c0, _L)] = m2a
            m1s[r, pl.ds(c0 + _L, _L)] = m1b
            m2s[r, pl.ds(c0 + _L, _L)] = m2b
            return _

        lax.fori_loop(0, _W // 32, col_body, None)


def _sc_body(vol, out, *scratch):
    bufs = scratch[:_NBUF]
    m1s, m2s, obuf = scratch[_NBUF:_NBUF + 3]
    sems = scratch[_NBUF + 3:]
    wid = lax.axis_index("s") * _NC + lax.axis_index("c")
    row0 = wid * _ROWS_PER_W
    b = row0 // _HS
    h0 = row0 % _HS

    def src(step):
        item = step // _NCH
        chunk = step % _NCH
        return vol.at[b, pl.ds(chunk * _DC, _DC), pl.ds(h0 + item * _RI, _RI), :]

    def start(step, k):
        pltpu.async_copy(src(step), bufs[k], sems[k])

    def wait(step, k):
        pltpu.make_async_copy(src(step), bufs[k], sems[k]).wait()

    def process(step, k):
        item = step // _NCH
        chunk = step % _NCH

        @pl.when(chunk == 0)
        def _():
            inf = jnp.full((_L,), _INF, jnp.float32)
            for r in range(_RI):
                def init_body(jj, _, r=r):
                    m1s[r, pl.ds(jj * _L, _L)] = inf
                    m2s[r, pl.ds(jj * _L, _L)] = inf
                    return _
                lax.fori_loop(0, _W // _L, init_body, None)

        wait(step, k)
        _compute_chunk(bufs[k], m1s, m2s)

        @pl.when(chunk == _NCH - 1)
        def _():
            for r in range(_RI):
                def out_body(jj, _, r=r):
                    sl = pl.ds(jj * _L, _L)
                    obuf[r, sl] = m2s[r, sl] - m1s[r, sl]
                    return _
                lax.fori_loop(0, _W // _L, out_body, None)
            pltpu.sync_copy(obuf, out.at[pl.ds(row0 + item * _RI, _RI), :])

    for k in range(_NBUF - 1):
        start(k, k)

    def loop_body(i, _):
        s0 = _NBUF * i
        for k in range(_NBUF):
            nxt = s0 + k + _NBUF - 1

            @pl.when(nxt < _STEPS)
            def _(nxt=nxt, k=k):
                start(nxt, (k + _NBUF - 1) % _NBUF)

            process(s0 + k, k)
        return _

    lax.fori_loop(0, _STEPS // _NBUF, loop_body, None)


def _sc_call(volume):
    mesh = plsc.VectorSubcoreMesh(core_axis_name="c", subcore_axis_name="s")
    return pl.kernel(
        _sc_body,
        out_type=jax.ShapeDtypeStruct((_SC_ROWS, _W), jnp.float32),
        mesh=mesh,
        scratch_types=(
            [pltpu.VMEM((_DC, _RI, _W), jnp.float32)] * _NBUF
            + [pltpu.VMEM((_RI, _W), jnp.float32)] * 3
            + [pltpu.SemaphoreType.DMA] * _NBUF
        ),
    )(volume)


def _tc_body(v_ref, o_ref):
    d = v_ref.shape[1]
    a = v_ref[0, 0]
    b = v_ref[0, 1]
    m1 = jnp.minimum(a, b)
    m2 = jnp.maximum(a, b)
    for i in range(2, d):
        x = v_ref[0, i]
        m2 = jnp.minimum(m2, jnp.maximum(m1, x))
        m1 = jnp.minimum(m1, x)
    o_ref[0, 0] = m2 - m1


def _tc_call(volume):
    nh = (_H - _HS) // _TH
    off = _HS // _TH
    return pl.pallas_call(
        _tc_body,
        grid=(_B, nh),
        in_specs=[pl.BlockSpec((1, _D, _TH, _W), lambda i, j: (i, 0, j + off, 0))],
        out_specs=pl.BlockSpec((1, 1, _TH, _W), lambda i, j: (i, 0, j, 0)),
        out_shape=jax.ShapeDtypeStruct((_B, 1, _H - _HS, _W), volume.dtype),
    )(volume)


@jax.jit
def _hybrid(volume):
    sc = _sc_call(volume).reshape(_B, 1, _HS, _W)
    if _HS == _H:
        return sc
    tc = _tc_call(volume)
    return jnp.concatenate([sc, tc], axis=2)


def kernel(volume):
    return _hybrid(volume)


# speedup vs baseline: 1.6305x; 1.0108x over previous
import functools

import jax
import jax.numpy as jnp
from jax import lax
from jax.experimental import pallas as pl
from jax.experimental.pallas import tpu as pltpu
from jax.experimental.pallas import tpu_sc as plsc

_B, _D, _H, _W = 2, 128, 512, 512
_HS = 64
_TH = 32

_NC, _NS, _L = 2, 16, 16
_NW = _NC * _NS
_RI = 4
_DC = 16
_NBUF = 2
_NCH = _D // _DC
_SC_ROWS = _B * _HS
_ROWS_PER_W = _SC_ROWS // _NW
_ITEMS = _ROWS_PER_W // _RI
_STEPS = _ITEMS * _NCH
_INF = float("inf")


def _compute_chunk(buf, m1s, m2s):
    for r in range(_RI):
        def col_body(jj, _, r=r):
            c0 = jj * 32
            m1a = m1s[r, pl.ds(c0, _L)]
            m2a = m2s[r, pl.ds(c0, _L)]
            m1b = m1s[r, pl.ds(c0 + _L, _L)]
            m2b = m2s[r, pl.ds(c0 + _L, _L)]
            for dd in range(_DC):
                xa = buf[dd, r, pl.ds(c0, _L)]
                xb = buf[dd, r, pl.ds(c0 + _L, _L)]
                m2a = jnp.minimum(m2a, jnp.maximum(m1a, xa))
                m1a = jnp.minimum(m1a, xa)
                m2b = jnp.minimum(m2b, jnp.maximum(m1b, xb))
                m1b = jnp.minimum(m1b, xb)
            m1s[r, pl.ds(c0, _L)] = m1a
            m2s[r, pl.ds(c0, _L)] = m2a
            m1s[r, pl.ds(c0 + _L, _L)] = m1b
            m2s[r, pl.ds(c0 + _L, _L)] = m2b
            return _

        lax.fori_loop(0, _W // 32, col_body, None)


def _sc_body(vol, out, *scratch):
    bufs = scratch[:_NBUF]
    m1s, m2s, obuf = scratch[_NBUF:_NBUF + 3]
    sems = scratch[_NBUF + 3:]
    wid = lax.axis_index("s") * _NC + lax.axis_index("c")
    row0 = wid * _ROWS_PER_W
    b = row0 // _HS
    h0 = row0 % _HS

    def src(step):
        item = step // _NCH
        chunk = step % _NCH
        return vol.at[b, pl.ds(chunk * _DC, _DC), pl.ds(h0 + item * _RI, _RI), :]

    def start(step, k):
        pltpu.async_copy(src(step), bufs[k], sems[k])

    def wait(step, k):
        pltpu.make_async_copy(src(step), bufs[k], sems[k]).wait()

    def process(step, k):
        item = step // _NCH
        chunk = step % _NCH

        @pl.when(chunk == 0)
        def _():
            inf = jnp.full((_L,), _INF, jnp.float32)
            for r in range(_RI):
                def init_body(jj, _, r=r):
                    m1s[r, pl.ds(jj * _L, _L)] = inf
                    m2s[r, pl.ds(jj * _L, _L)] = inf
                    return _
                lax.fori_loop(0, _W // _L, init_body, None)

        wait(step, k)
        _compute_chunk(bufs[k], m1s, m2s)

        @pl.when(chunk == _NCH - 1)
        def _():
            for r in range(_RI):
                def out_body(jj, _, r=r):
                    sl = pl.ds(jj * _L, _L)
                    obuf[r, sl] = m2s[r, sl] - m1s[r, sl]
                    return _
                lax.fori_loop(0, _W // _L, out_body, None)
            pltpu.sync_copy(obuf, out.at[pl.ds(row0 + item * _RI, _RI), :])

    for k in range(_NBUF - 1):
        start(k, k)

    def loop_body(i, _):
        s0 = _NBUF * i
        for k in range(_NBUF):
            nxt = s0 + k + _NBUF - 1

            @pl.when(nxt < _STEPS)
            def _(nxt=nxt, k=k):
                start(nxt, (k + _NBUF - 1) % _NBUF)

            process(s0 + k, k)
        return _

    lax.fori_loop(0, _STEPS // _NBUF, loop_body, None)


def _sc_call(volume):
    mesh = plsc.VectorSubcoreMesh(core_axis_name="c", subcore_axis_name="s")
    return pl.kernel(
        _sc_body,
        out_type=jax.ShapeDtypeStruct((_SC_ROWS, _W), jnp.float32),
        mesh=mesh,
        scratch_types=(
            [pltpu.VMEM((_DC, _RI, _W), jnp.float32)] * _NBUF
            + [pltpu.VMEM((_RI, _W), jnp.float32)] * 3
            + [pltpu.SemaphoreType.DMA] * _NBUF
        ),
    )(volume)


def _tc_body(v_ref, o_ref):
    d = v_ref.shape[1]
    a = v_ref[0, 0]
    b = v_ref[0, 1]
    m1 = jnp.minimum(a, b)
    m2 = jnp.maximum(a, b)
    for i in range(2, d):
        x = v_ref[0, i]
        m2 = jnp.minimum(m2, jnp.maximum(m1, x))
        m1 = jnp.minimum(m1, x)
    o_ref[0, 0] = m2 - m1


def _tc_call(volume):
    nh = (_H - _HS) // _TH
    off = _HS // _TH
    return pl.pallas_call(
        _tc_body,
        grid=(_B, nh),
        in_specs=[pl.BlockSpec((1, _D, _TH, _W), lambda i, j: (i, 0, j + off, 0))],
        out_specs=pl.BlockSpec((1, 1, _TH, _W), lambda i, j: (i, 0, j, 0)),
        out_shape=jax.ShapeDtypeStruct((_B, 1, _H - _HS, _W), volume.dtype),
    )(volume)


@jax.jit
def _hybrid(volume):
    sc = _sc_call(volume).reshape(_B, 1, _HS, _W)
    if _HS == _H:
        return sc
    tc = _tc_call(volume)
    return jnp.concatenate([sc, tc], axis=2)


def kernel(volume):
    return _hybrid(volume)
